# Initial kernel scaffold; baseline (speedup 1.0000x reference)
import functools

import jax
import jax.numpy as jnp
from jax import lax
from jax.experimental import pallas as pl
from jax.experimental.pallas import tpu as pltpu
from jax.experimental.pallas import tpu_sc as plsc

N_NEG = 5
D = 128
LANES = 16
CHUNK = 80
NCHUNK = N_NEG * CHUNK
GROUPS = CHUNK // LANES
NC = 2
NS = 16
NW = NC * NS


def _make_sc_kernel(n_edges):
    per_w = n_edges // NW
    n_chunks = per_w // CHUNK
    assert per_w * NW == n_edges and n_chunks * CHUNK == per_w

    mesh = plsc.VectorSubcoreMesh(
        core_axis_name="c", subcore_axis_name="s",
        num_cores=NC, num_subcores=NS)

    @functools.partial(
        pl.kernel,
        mesh=mesh,
        out_type=(
            jax.ShapeDtypeStruct((n_edges,), jnp.float32),
            jax.ShapeDtypeStruct((N_NEG * n_edges,), jnp.float32),
        ),
        scratch_types=(
            pltpu.VMEM((CHUNK,), jnp.int32),
            pltpu.VMEM((CHUNK,), jnp.int32),
            pltpu.VMEM((N_NEG, CHUNK), jnp.int32),
            pltpu.VMEM((CHUNK, D), jnp.float32),
            pltpu.VMEM((CHUNK, D), jnp.float32),
            pltpu.VMEM((NCHUNK, D), jnp.float32),
            pltpu.VMEM((CHUNK,), jnp.float32),
            pltpu.VMEM((NCHUNK,), jnp.float32),
            pltpu.SemaphoreType.DMA,
        ),
    )
    def k(z_hbm, src_hbm, dst_hbm, neg_hbm, pos_out, neg_out,
          sidx, didx, nidx, s_rows, d_rows, n_rows, pos_stage, neg_stage,
          sem):
        wid = lax.axis_index("s") * NC + lax.axis_index("c")
        wbase = wid * per_w

        def chunk_body(i, carry):
            base = pl.multiple_of(wbase + i * CHUNK, 8)
            nbase = pl.multiple_of(base * N_NEG, 8)

            pltpu.sync_copy(src_hbm.at[pl.ds(base, CHUNK)], sidx)
            pltpu.sync_copy(dst_hbm.at[pl.ds(base, CHUNK)], didx)
            for kk in range(N_NEG):
                pltpu.sync_copy(
                    neg_hbm.at[pl.ds(nbase + kk * CHUNK, CHUNK)], nidx.at[kk])

            cps = [
                pltpu.async_copy(z_hbm.at[sidx], s_rows, sem),
                pltpu.async_copy(z_hbm.at[didx], d_rows, sem),
            ]
            for kk in range(N_NEG):
                cps.append(pltpu.async_copy(
                    z_hbm.at[nidx.at[kk]],
                    n_rows.at[pl.ds(kk * CHUNK, CHUNK)], sem))
            for cp in cps:
                cp.wait()

            def group_body(g, gc):
                elanes = g * LANES + lax.iota(jnp.int32, LANES)
                nrow0 = elanes * N_NEG
                accs = [jnp.zeros((LANES,), jnp.float32)
                        for _ in range(1 + N_NEG)]
                for f in range(D):
                    fsp = jnp.full((LANES,), f, jnp.int32)
                    s = plsc.load_gather(s_rows, [elanes, fsp])
                    dv = plsc.load_gather(d_rows, [elanes, fsp])
                    accs[0] = accs[0] + s * dv
                    for kk in range(N_NEG):
                        nv = plsc.load_gather(n_rows, [nrow0 + kk, fsp])
                        accs[1 + kk] = accs[1 + kk] + s * nv
                pos_stage[pl.ds(g * LANES, LANES)] = (
                    1.0 / (1.0 + jnp.exp(-accs[0])))
                for kk in range(N_NEG):
                    plsc.store_scatter(
                        neg_stage, [nrow0 + kk],
                        1.0 / (1.0 + jnp.exp(-accs[1 + kk])))
                return gc

            lax.fori_loop(0, GROUPS, group_body, 0)
            pltpu.sync_copy(pos_stage, pos_out.at[pl.ds(base, CHUNK)])
            pltpu.sync_copy(neg_stage, neg_out.at[pl.ds(nbase, NCHUNK)])
            return carry

        lax.fori_loop(0, n_chunks, chunk_body, 0)

    return k


def kernel(z, pos_edge_index, n_nodes):
    n_edges = pos_edge_index.shape[1]
    neg_dst = jax.random.randint(
        jax.random.key(42), (n_edges * N_NEG,), 0, n_nodes, dtype=jnp.int32)
    src = pos_edge_index[0]
    dst = pos_edge_index[1]
    pos_scores, neg_scores = _make_sc_kernel(n_edges)(z, src, dst, neg_dst)
    return (pos_scores, neg_scores)


# speedup vs baseline: 1.7347x; 1.7347x over previous
import functools

import jax
import jax.numpy as jnp
from jax import lax
from jax.experimental import pallas as pl
from jax.experimental.pallas import tpu as pltpu
from jax.experimental.pallas import tpu_sc as plsc

N_NEG = 5
D = 128
LANES = 16
CHUNK = 80
NCHUNK = N_NEG * CHUNK
GROUPS = CHUNK // LANES
NC = 2
NS = 16
NW = NC * NS


def _make_sc_kernel(n_edges):
    per_w = n_edges // NW
    n_chunks = per_w // CHUNK
    assert per_w * NW == n_edges and n_chunks * CHUNK == per_w

    mesh = plsc.VectorSubcoreMesh(
        core_axis_name="c", subcore_axis_name="s",
        num_cores=NC, num_subcores=NS)

    @functools.partial(
        pl.kernel,
        mesh=mesh,
        out_type=(
            jax.ShapeDtypeStruct((n_edges,), jnp.float32),
            jax.ShapeDtypeStruct((N_NEG * n_edges,), jnp.float32),
        ),
        scratch_types=(
            pltpu.VMEM((CHUNK,), jnp.int32),
            pltpu.VMEM((CHUNK,), jnp.int32),
            pltpu.VMEM((N_NEG, CHUNK), jnp.int32),
            pltpu.VMEM((CHUNK, D), jnp.float32),
            pltpu.VMEM((CHUNK, D), jnp.float32),
            pltpu.VMEM((NCHUNK, D), jnp.float32),
            pltpu.VMEM((CHUNK,), jnp.float32),
            pltpu.VMEM((NCHUNK,), jnp.float32),
            pltpu.SemaphoreType.DMA,
        ),
        compiler_params=pltpu.CompilerParams(needs_layout_passes=False),
    )
    def k(z_hbm, src_hbm, dst_hbm, neg_hbm, pos_out, neg_out,
          sidx, didx, nidx, s_rows, d_rows, n_rows, pos_stage, neg_stage,
          sem):
        wid = lax.axis_index("s") * NC + lax.axis_index("c")
        wbase = wid * per_w

        def chunk_body(i, carry):
            base = pl.multiple_of(wbase + i * CHUNK, 8)
            nbase = pl.multiple_of(base * N_NEG, 8)

            pltpu.sync_copy(src_hbm.at[pl.ds(base, CHUNK)], sidx)
            pltpu.sync_copy(dst_hbm.at[pl.ds(base, CHUNK)], didx)
            for kk in range(N_NEG):
                pltpu.sync_copy(
                    neg_hbm.at[pl.ds(nbase + kk * CHUNK, CHUNK)], nidx.at[kk])

            cps = [
                pltpu.async_copy(z_hbm.at[sidx], s_rows, sem),
                pltpu.async_copy(z_hbm.at[didx], d_rows, sem),
            ]
            for kk in range(N_NEG):
                cps.append(pltpu.async_copy(
                    z_hbm.at[nidx.at[kk]],
                    n_rows.at[pl.ds(kk * CHUNK, CHUNK)], sem))
            for cp in cps:
                cp.wait()

            def group_body(g, gc):
                elanes = g * LANES + lax.iota(jnp.int32, LANES)
                nrow0 = elanes * N_NEG
                accs = [jnp.zeros((LANES,), jnp.float32)
                        for _ in range(1 + N_NEG)]
                for f in range(D):
                    fsp = jnp.full((LANES,), f, jnp.int32)
                    s = plsc.load_gather(s_rows, [elanes, fsp])
                    dv = plsc.load_gather(d_rows, [elanes, fsp])
                    accs[0] = accs[0] + s * dv
                    for kk in range(N_NEG):
                        nv = plsc.load_gather(n_rows, [nrow0 + kk, fsp])
                        accs[1 + kk] = accs[1 + kk] + s * nv
                pos_stage[pl.ds(g * LANES, LANES)] = (
                    1.0 / (1.0 + jnp.exp(-accs[0])))
                for kk in range(N_NEG):
                    plsc.store_scatter(
                        neg_stage, [nrow0 + kk],
                        1.0 / (1.0 + jnp.exp(-accs[1 + kk])))
                return gc

            lax.fori_loop(0, GROUPS, group_body, 0)
            pltpu.sync_copy(pos_stage, pos_out.at[pl.ds(base, CHUNK)])
            pltpu.sync_copy(neg_stage, neg_out.at[pl.ds(nbase, NCHUNK)])
            return carry

        lax.fori_loop(0, n_chunks, chunk_body, 0)

    return k


def kernel(z, pos_edge_index, n_nodes):
    n_edges = pos_edge_index.shape[1]
    neg_dst = jax.random.randint(
        jax.random.key(42), (n_edges * N_NEG,), 0, n_nodes, dtype=jnp.int32)
    src = pos_edge_index[0]
    dst = pos_edge_index[1]
    pos_scores, neg_scores = _make_sc_kernel(n_edges)(z, src, dst, neg_dst)
    return (pos_scores, neg_scores)


# speedup vs baseline: 2.0054x; 1.1560x over previous
import functools

import jax
import jax.numpy as jnp
from jax import lax
from jax.experimental import pallas as pl
from jax.experimental.pallas import tpu as pltpu
from jax.experimental.pallas import tpu_sc as plsc

N_NEG = 5
D = 128
LANES = 16
CHUNK = 64
NCHUNK = N_NEG * CHUNK
ROWS = 2 * CHUNK + NCHUNK
GROUPS = CHUNK // LANES
IB = 8
FUNROLL = 16
NC = 2
NS = 16
NW = NC * NS

_SEGS = []
_off = 0
while _off < ROWS:
    _len = min(128, ROWS - _off)
    _SEGS.append((_off, _len))
    _off += _len


def _pack_indices(src, dst, neg_dst, per_w, ncht, nreg):
    def chunkify(a, cw):
        a = a.reshape(NW, -1)
        reg = a[:, :nreg * cw].reshape(NW, nreg, cw)
        parts = [reg]
        if ncht > nreg:
            tail = a[:, a.shape[1] - cw:][:, None, :]
            parts.append(jnp.broadcast_to(tail, (NW, ncht - nreg, cw)))
        return jnp.concatenate(parts, axis=1) if len(parts) > 1 else reg

    return jnp.concatenate(
        [chunkify(src, CHUNK), chunkify(dst, CHUNK),
         chunkify(neg_dst, NCHUNK)], axis=2)


def _make_sc_kernel(n_edges):
    per_w = n_edges // NW
    assert per_w * NW == n_edges
    assert per_w >= CHUNK and (per_w - CHUNK) % 8 == 0
    nreg = per_w // CHUNK
    covered = nreg * CHUNK
    ncht = ((nreg + (1 if covered < per_w else 0) + IB - 1) // IB) * IB
    npair = ncht // 2
    tail_start = per_w - CHUNK

    mesh = plsc.VectorSubcoreMesh(
        core_axis_name="c", subcore_axis_name="s",
        num_cores=NC, num_subcores=NS)

    @functools.partial(
        pl.kernel,
        mesh=mesh,
        out_type=(
            jax.ShapeDtypeStruct((n_edges,), jnp.float32),
            jax.ShapeDtypeStruct((N_NEG * n_edges,), jnp.float32),
        ),
        scratch_types=(
            pltpu.VMEM((2, IB, ROWS), jnp.int32),
            pltpu.VMEM((ROWS, D), jnp.float32),
            pltpu.VMEM((ROWS, D), jnp.float32),
            pltpu.VMEM((CHUNK + NCHUNK,), jnp.float32),
            pltpu.VMEM((CHUNK + NCHUNK,), jnp.float32),
            pltpu.SemaphoreType.DMA,
            pltpu.SemaphoreType.DMA,
            pltpu.SemaphoreType.DMA,
            pltpu.SemaphoreType.DMA,
        ),
        compiler_params=pltpu.CompilerParams(needs_layout_passes=False),
    )
    def k(z_hbm, idx_hbm, pos_out, neg_out,
          idx_blk, rows0, rows1, stage0, stage1,
          gsem0, gsem1, osem0, osem1):
        wid = lax.axis_index("s") * NC + lax.axis_index("c")
        wbase = wid * per_w

        def issue_gathers(c, rows_q, gsem_q):
            slot = lax.rem(lax.div(c, IB), 2)
            row = lax.rem(c, IB)
            for off, ln in _SEGS:
                pltpu.async_copy(
                    z_hbm.at[idx_blk.at[slot, row, pl.ds(off, ln)]],
                    rows_q.at[pl.ds(off, ln)], gsem_q)

        def wait_gathers(rows_q, gsem_q):
            for off, ln in _SEGS:
                pltpu.make_async_copy(
                    z_hbm.at[idx_blk.at[0, 0, pl.ds(off, ln)]],
                    rows_q.at[pl.ds(off, ln)], gsem_q).wait()

        def wait_outs(c, stage_q, osem_q):
            gb = pl.multiple_of(
                wbase + jnp.minimum(c * CHUNK, tail_start), 8)
            pltpu.make_async_copy(
                stage_q.at[pl.ds(0, CHUNK)],
                pos_out.at[pl.ds(gb, CHUNK)], osem_q).wait()
            pltpu.make_async_copy(
                stage_q.at[pl.ds(CHUNK, NCHUNK)],
                neg_out.at[pl.ds(gb * N_NEG, NCHUNK)], osem_q).wait()

        def compute(rows_q, stage_q):
            def group_body(g, gc):
                elanes = g * LANES + lax.iota(jnp.int32, LANES)
                drow = CHUNK + elanes
                nrow0 = 2 * CHUNK + elanes * N_NEG

                def feat_body(fb, accs):
                    accs = list(accs)
                    for j in range(FUNROLL):
                        f = fb * FUNROLL + j
                        fsp = jnp.full((LANES,), 1, jnp.int32) * f
                        s = plsc.load_gather(rows_q, [elanes, fsp])
                        dv = plsc.load_gather(rows_q, [drow, fsp])
                        accs[0] = accs[0] + s * dv
                        for kk in range(N_NEG):
                            nv = plsc.load_gather(rows_q, [nrow0 + kk, fsp])
                            accs[1 + kk] = accs[1 + kk] + s * nv
                    return tuple(accs)

                zero = jnp.zeros((LANES,), jnp.float32)
                accs = lax.fori_loop(
                    0, D // FUNROLL, feat_body, (zero,) * (1 + N_NEG))
                stage_q[pl.ds(g * LANES, LANES)] = (
                    1.0 / (1.0 + jnp.exp(-accs[0])))
                nidx0 = CHUNK + elanes * N_NEG
                for kk in range(N_NEG):
                    plsc.store_scatter(
                        stage_q, [nidx0 + kk],
                        1.0 / (1.0 + jnp.exp(-accs[1 + kk])))
                return gc

            lax.fori_loop(0, GROUPS, group_body, 0)

        def issue_outs(c, stage_q, osem_q):
            gb = pl.multiple_of(
                wbase + jnp.minimum(c * CHUNK, tail_start), 8)
            pltpu.async_copy(stage_q.at[pl.ds(0, CHUNK)],
                             pos_out.at[pl.ds(gb, CHUNK)], osem_q)
            pltpu.async_copy(stage_q.at[pl.ds(CHUNK, NCHUNK)],
                             neg_out.at[pl.ds(gb * N_NEG, NCHUNK)], osem_q)

        pltpu.sync_copy(idx_hbm.at[wid, pl.ds(0, IB)], idx_blk.at[0])
        issue_gathers(0, rows0, gsem0)

        def pair_body(i, carry):
            c0 = 2 * i
            c1 = c0 + 1

            @pl.when(c0 >= 2)
            def _():
                wait_outs(c0 - 2, stage0, osem0)
            issue_gathers(c1, rows1, gsem1)
            wait_gathers(rows0, gsem0)
            compute(rows0, stage0)
            issue_outs(c0, stage0, osem0)

            @pl.when(c1 >= 2)
            def _():
                wait_outs(c1 - 2, stage1, osem1)

            @pl.when(jnp.logical_and(lax.rem(c1 + 1, IB) == 0,
                                     c1 + 1 < ncht))
            def _():
                blk = lax.div(c1 + 1, IB)
                pltpu.sync_copy(
                    idx_hbm.at[wid, pl.ds(blk * IB, IB)],
                    idx_blk.at[lax.rem(blk, 2)])

            @pl.when(c1 + 1 < ncht)
            def _():
                issue_gathers(c1 + 1, rows0, gsem0)
            wait_gathers(rows1, gsem1)
            compute(rows1, stage1)
            issue_outs(c1, stage1, osem1)
            return carry

        lax.fori_loop(0, npair, pair_body, 0)
        wait_outs(ncht - 2, stage0, osem0)
        wait_outs(ncht - 1, stage1, osem1)

    return k, per_w, ncht, nreg


def kernel(z, pos_edge_index, n_nodes):
    n_edges = pos_edge_index.shape[1]
    neg_dst = jax.random.randint(
        jax.random.key(42), (n_edges * N_NEG,), 0, n_nodes, dtype=jnp.int32)
    src = pos_edge_index[0]
    dst = pos_edge_index[1]
    k, per_w, ncht, nreg = _make_sc_kernel(n_edges)
    idx_pack = _pack_indices(src, dst, neg_dst, per_w, ncht, nreg)
    pos_scores, neg_scores = k(z, idx_pack)
    return (pos_scores, neg_scores)


# speedup vs baseline: 5.9060x; 2.9450x over previous
import functools

import jax
import jax.numpy as jnp
from jax import lax
from jax.experimental import pallas as pl
from jax.experimental.pallas import tpu as pltpu
from jax.experimental.pallas import tpu_sc as plsc

N_NEG = 5
D = 128
LANES = 16
CHUNK = 64
NCHUNK = N_NEG * CHUNK
ROWS = 2 * CHUNK + NCHUNK
GROUPS = CHUNK // LANES
IB = 8
FUNROLL = 16
NC = 2
NS = 16
NW = NC * NS

_SEGS = []
_off = 0
while _off < ROWS:
    _len = min(128, ROWS - _off)
    _SEGS.append((_off, _len))
    _off += _len


def _pack_indices(src, dst, neg_dst, per_w, ncht, nreg):
    def chunkify(a, cw):
        a = a.reshape(NW, -1)
        reg = a[:, :nreg * cw].reshape(NW, nreg, cw)
        parts = [reg]
        if ncht > nreg:
            tail = a[:, a.shape[1] - cw:][:, None, :]
            parts.append(jnp.broadcast_to(tail, (NW, ncht - nreg, cw)))
        return jnp.concatenate(parts, axis=1) if len(parts) > 1 else reg

    return jnp.concatenate(
        [chunkify(src, CHUNK), chunkify(dst, CHUNK),
         chunkify(neg_dst, NCHUNK)], axis=2)


def _make_sc_kernel(n_edges):
    per_w = n_edges // NW
    assert per_w * NW == n_edges
    assert per_w >= CHUNK and (per_w - CHUNK) % 8 == 0
    nreg = per_w // CHUNK
    covered = nreg * CHUNK
    ncht = ((nreg + (1 if covered < per_w else 0) + IB - 1) // IB) * IB
    npair = ncht // 2
    tail_start = per_w - CHUNK

    mesh = plsc.VectorSubcoreMesh(
        core_axis_name="c", subcore_axis_name="s",
        num_cores=NC, num_subcores=NS)

    @functools.partial(
        pl.kernel,
        mesh=mesh,
        out_type=(
            jax.ShapeDtypeStruct((n_edges,), jnp.float32),
            jax.ShapeDtypeStruct((N_NEG * n_edges,), jnp.float32),
        ),
        scratch_types=(
            pltpu.VMEM((ROWS,), jnp.int32),
            pltpu.VMEM((ROWS,), jnp.int32),
            pltpu.VMEM((ROWS, D), jnp.float32),
            pltpu.VMEM((ROWS, D), jnp.float32),
            pltpu.VMEM((CHUNK + NCHUNK,), jnp.float32),
            pltpu.VMEM((CHUNK + NCHUNK,), jnp.float32),
            pltpu.VMEM(((1 + N_NEG) * LANES, LANES), jnp.float32),
            pltpu.SemaphoreType.DMA,
            pltpu.SemaphoreType.DMA,
            pltpu.SemaphoreType.DMA,
            pltpu.SemaphoreType.DMA,
            pltpu.SemaphoreType.DMA,
            pltpu.SemaphoreType.DMA,
        ),
        compiler_params=pltpu.CompilerParams(needs_layout_passes=False),
    )
    def k(z_hbm, idx_hbm, pos_out, neg_out,
          idxb0, idxb1, rows0, rows1, stage0, stage1, tbuf,
          gsem0, gsem1, osem0, osem1, isem0, isem1):
        wid = lax.axis_index("s") * NC + lax.axis_index("c")
        wbase = wid * per_w

        def issue_gathers(idxb_q, rows_q, gsem_q):
            for off, ln in _SEGS:
                pltpu.async_copy(
                    z_hbm.at[idxb_q.at[pl.ds(off, ln)]],
                    rows_q.at[pl.ds(off, ln)], gsem_q)

        def wait_gathers(idxb_q, rows_q, gsem_q):
            for off, ln in _SEGS:
                pltpu.make_async_copy(
                    z_hbm.at[idxb_q.at[pl.ds(off, ln)]],
                    rows_q.at[pl.ds(off, ln)], gsem_q).wait()

        def issue_idx(c, idxb_q, isem_q):
            pltpu.async_copy(idx_hbm.at[wid, c], idxb_q, isem_q)

        def wait_idx(idxb_q, isem_q):
            pltpu.make_async_copy(idx_hbm.at[wid, 0], idxb_q, isem_q).wait()

        def wait_outs(c, stage_q, osem_q):
            gb = pl.multiple_of(
                wbase + jnp.minimum(c * CHUNK, tail_start), 8)
            pltpu.make_async_copy(
                stage_q.at[pl.ds(0, CHUNK)],
                pos_out.at[pl.ds(gb, CHUNK)], osem_q).wait()
            pltpu.make_async_copy(
                stage_q.at[pl.ds(CHUNK, NCHUNK)],
                neg_out.at[pl.ds(gb * N_NEG, NCHUNK)], osem_q).wait()

        def compute(rows_q, stage_q, tb):
            iota = lax.iota(jnp.int32, LANES)

            def group_body(g, gc):
                ebase = g * LANES

                def edge_body(el, ec):
                    e = ebase + el
                    s_regs = [rows_q[e, pl.ds(16 * j, 16)] for j in range(8)]

                    def rowdot(prow):
                        t = [s_regs[j] * rows_q[prow, pl.ds(16 * j, 16)]
                             for j in range(8)]
                        t = [t[0] + t[1], t[2] + t[3],
                             t[4] + t[5], t[6] + t[7]]
                        t = [t[0] + t[1], t[2] + t[3]]
                        return t[0] + t[1]

                    tb[el, :] = rowdot(CHUNK + e)
                    for kk in range(N_NEG):
                        tb[(1 + kk) * LANES + el, :] = rowdot(
                            2 * CHUNK + N_NEG * e + kk)
                    return ec

                lax.fori_loop(0, LANES, edge_body, 0)

                def colsum(b):
                    rvec = jnp.full((LANES,), b * LANES, jnp.int32) + iota
                    gs = [plsc.load_gather(
                        tb, [rvec, jnp.full((LANES,), j, jnp.int32)])
                        for j in range(LANES)]
                    for step in (8, 4, 2, 1):
                        gs = [gs[i] + gs[i + step] for i in range(step)]
                    return gs[0]

                stage_q[pl.ds(ebase, LANES)] = (
                    1.0 / (1.0 + jnp.exp(-colsum(0))))
                nidx0 = CHUNK + (ebase + iota) * N_NEG
                for kk in range(N_NEG):
                    plsc.store_scatter(
                        stage_q, [nidx0 + kk],
                        1.0 / (1.0 + jnp.exp(-colsum(1 + kk))))
                return gc

            lax.fori_loop(0, GROUPS, group_body, 0)

        def issue_outs(c, stage_q, osem_q):
            gb = pl.multiple_of(
                wbase + jnp.minimum(c * CHUNK, tail_start), 8)
            pltpu.async_copy(stage_q.at[pl.ds(0, CHUNK)],
                             pos_out.at[pl.ds(gb, CHUNK)], osem_q)
            pltpu.async_copy(stage_q.at[pl.ds(CHUNK, NCHUNK)],
                             neg_out.at[pl.ds(gb * N_NEG, NCHUNK)], osem_q)

        pltpu.sync_copy(idx_hbm.at[wid, 0], idxb0)
        issue_gathers(idxb0, rows0, gsem0)
        issue_idx(1, idxb1, isem1)

        def pair_body(i, carry):
            c0 = 2 * i
            c1 = c0 + 1

            @pl.when(c0 >= 2)
            def _():
                wait_outs(c0 - 2, stage0, osem0)
            wait_idx(idxb1, isem1)
            issue_gathers(idxb1, rows1, gsem1)
            wait_gathers(idxb0, rows0, gsem0)

            @pl.when(c0 + 2 < ncht)
            def _():
                issue_idx(c0 + 2, idxb0, isem0)
            compute(rows0, stage0, tbuf)
            issue_outs(c0, stage0, osem0)

            @pl.when(c1 >= 2)
            def _():
                wait_outs(c1 - 2, stage1, osem1)

            @pl.when(c1 + 1 < ncht)
            def _():
                wait_idx(idxb0, isem0)
                issue_gathers(idxb0, rows0, gsem0)
            wait_gathers(idxb1, rows1, gsem1)

            @pl.when(c1 + 2 < ncht)
            def _():
                issue_idx(c1 + 2, idxb1, isem1)
            compute(rows1, stage1, tbuf)
            issue_outs(c1, stage1, osem1)
            return carry

        lax.fori_loop(0, npair, pair_body, 0)
        wait_outs(ncht - 2, stage0, osem0)
        wait_outs(ncht - 1, stage1, osem1)

    return k, per_w, ncht, nreg


def kernel(z, pos_edge_index, n_nodes):
    n_edges = pos_edge_index.shape[1]
    neg_dst = jax.random.randint(
        jax.random.key(42), (n_edges * N_NEG,), 0, n_nodes, dtype=jnp.int32)
    src = pos_edge_index[0]
    dst = pos_edge_index[1]
    k, per_w, ncht, nreg = _make_sc_kernel(n_edges)
    idx_pack = _pack_indices(src, dst, neg_dst, per_w, ncht, nreg)
    pos_scores, neg_scores = k(z, idx_pack)
    return (pos_scores, neg_scores)


# speedup vs baseline: 5.9321x; 1.0044x over previous
import functools

import jax
import jax.numpy as jnp
from jax import lax
from jax.experimental import pallas as pl
from jax.experimental.pallas import tpu as pltpu
from jax.experimental.pallas import tpu_sc as plsc

N_NEG = 5
D = 128
LANES = 16
CHUNK = 64
NCHUNK = N_NEG * CHUNK
ROWS = 2 * CHUNK + NCHUNK
GROUPS = CHUNK // LANES
IB = 8
FUNROLL = 16
NC = 2
NS = 16
NW = NC * NS

_SEGS = []
_off = 0
while _off < ROWS:
    _len = min(128, ROWS - _off)
    _SEGS.append((_off, _len))
    _off += _len


def _pack_indices(src, dst, neg_dst, per_w, ncht, nreg):
    def chunkify(a, cw):
        a = a.reshape(NW, -1)
        reg = a[:, :nreg * cw].reshape(NW, nreg, cw)
        parts = [reg]
        if ncht > nreg:
            tail = a[:, a.shape[1] - cw:][:, None, :]
            parts.append(jnp.broadcast_to(tail, (NW, ncht - nreg, cw)))
        return jnp.concatenate(parts, axis=1) if len(parts) > 1 else reg

    return jnp.concatenate(
        [chunkify(src, CHUNK), chunkify(dst, CHUNK),
         chunkify(neg_dst, NCHUNK)], axis=2)


def _make_sc_kernel(n_edges):
    per_w = n_edges // NW
    assert per_w * NW == n_edges
    assert per_w >= CHUNK and (per_w - CHUNK) % 8 == 0
    nreg = per_w // CHUNK
    covered = nreg * CHUNK
    ncht = ((nreg + (1 if covered < per_w else 0) + IB - 1) // IB) * IB
    npair = ncht // 2
    tail_start = per_w - CHUNK

    mesh = plsc.VectorSubcoreMesh(
        core_axis_name="c", subcore_axis_name="s",
        num_cores=NC, num_subcores=NS)

    @functools.partial(
        pl.kernel,
        mesh=mesh,
        out_type=(
            jax.ShapeDtypeStruct((n_edges,), jnp.float32),
            jax.ShapeDtypeStruct((N_NEG * n_edges,), jnp.float32),
        ),
        scratch_types=(
            pltpu.VMEM((ROWS,), jnp.int32),
            pltpu.VMEM((ROWS,), jnp.int32),
            pltpu.VMEM((ROWS, D), jnp.float32),
            pltpu.VMEM((ROWS, D), jnp.float32),
            pltpu.VMEM((CHUNK + NCHUNK,), jnp.float32),
            pltpu.VMEM((CHUNK + NCHUNK,), jnp.float32),
            pltpu.VMEM(((1 + N_NEG) * LANES, LANES), jnp.float32),
            pltpu.SemaphoreType.DMA,
            pltpu.SemaphoreType.DMA,
            pltpu.SemaphoreType.DMA,
            pltpu.SemaphoreType.DMA,
            pltpu.SemaphoreType.DMA,
            pltpu.SemaphoreType.DMA,
        ),
        compiler_params=pltpu.CompilerParams(needs_layout_passes=False),
    )
    def k(z_hbm, idx_hbm, pos_out, neg_out,
          idxb0, idxb1, rows0, rows1, stage0, stage1, tbuf,
          gsem0, gsem1, osem0, osem1, isem0, isem1):
        wid = lax.axis_index("s") * NC + lax.axis_index("c")
        wbase = wid * per_w

        def issue_gathers(idxb_q, rows_q, gsem_q):
            for off, ln in _SEGS:
                pltpu.async_copy(
                    z_hbm.at[idxb_q.at[pl.ds(off, ln)]],
                    rows_q.at[pl.ds(off, ln)], gsem_q)

        def wait_gathers(idxb_q, rows_q, gsem_q):
            for off, ln in _SEGS:
                pltpu.make_async_copy(
                    z_hbm.at[idxb_q.at[pl.ds(off, ln)]],
                    rows_q.at[pl.ds(off, ln)], gsem_q).wait()

        def issue_idx(c, idxb_q, isem_q):
            pltpu.async_copy(idx_hbm.at[wid, c], idxb_q, isem_q)

        def wait_idx(idxb_q, isem_q):
            pltpu.make_async_copy(idx_hbm.at[wid, 0], idxb_q, isem_q).wait()

        def wait_outs(c, stage_q, osem_q):
            gb = pl.multiple_of(
                wbase + jnp.minimum(c * CHUNK, tail_start), 8)
            pltpu.make_async_copy(
                stage_q.at[pl.ds(0, CHUNK)],
                pos_out.at[pl.ds(gb, CHUNK)], osem_q).wait()
            pltpu.make_async_copy(
                stage_q.at[pl.ds(CHUNK, NCHUNK)],
                neg_out.at[pl.ds(gb * N_NEG, NCHUNK)], osem_q).wait()

        def compute(rows_q, stage_q, tb):
            iota = lax.iota(jnp.int32, LANES)

            def group_body(g, gc):
                ebase = g * LANES

                def edge_body(il, ec):
                    for u in range(4):
                        el = il * 4 + u
                        e = ebase + el
                        s_regs = [rows_q[e, pl.ds(16 * j, 16)]
                                  for j in range(8)]

                        def rowdot(prow, s_regs=s_regs):
                            t = [s_regs[j] * rows_q[prow, pl.ds(16 * j, 16)]
                                 for j in range(8)]
                            t = [t[0] + t[1], t[2] + t[3],
                                 t[4] + t[5], t[6] + t[7]]
                            t = [t[0] + t[1], t[2] + t[3]]
                            return t[0] + t[1]

                        tb[el, :] = rowdot(CHUNK + e)
                        for kk in range(N_NEG):
                            tb[(1 + kk) * LANES + el, :] = rowdot(
                                2 * CHUNK + N_NEG * e + kk)
                    return ec

                lax.fori_loop(0, LANES // 4, edge_body, 0)

                def colsum(b):
                    rvec = jnp.full((LANES,), b * LANES, jnp.int32) + iota
                    gs = [plsc.load_gather(
                        tb, [rvec, jnp.full((LANES,), j, jnp.int32)])
                        for j in range(LANES)]
                    for step in (8, 4, 2, 1):
                        gs = [gs[i] + gs[i + step] for i in range(step)]
                    return gs[0]

                stage_q[pl.ds(ebase, LANES)] = (
                    1.0 / (1.0 + jnp.exp(-colsum(0))))
                nidx0 = CHUNK + (ebase + iota) * N_NEG
                for kk in range(N_NEG):
                    plsc.store_scatter(
                        stage_q, [nidx0 + kk],
                        1.0 / (1.0 + jnp.exp(-colsum(1 + kk))))
                return gc

            lax.fori_loop(0, GROUPS, group_body, 0)

        def issue_outs(c, stage_q, osem_q):
            gb = pl.multiple_of(
                wbase + jnp.minimum(c * CHUNK, tail_start), 8)
            pltpu.async_copy(stage_q.at[pl.ds(0, CHUNK)],
                             pos_out.at[pl.ds(gb, CHUNK)], osem_q)
            pltpu.async_copy(stage_q.at[pl.ds(CHUNK, NCHUNK)],
                             neg_out.at[pl.ds(gb * N_NEG, NCHUNK)], osem_q)

        pltpu.sync_copy(idx_hbm.at[wid, 0], idxb0)
        issue_gathers(idxb0, rows0, gsem0)
        issue_idx(1, idxb1, isem1)

        def pair_body(i, carry):
            c0 = 2 * i
            c1 = c0 + 1

            @pl.when(c0 >= 2)
            def _():
                wait_outs(c0 - 2, stage0, osem0)
            wait_idx(idxb1, isem1)
            issue_gathers(idxb1, rows1, gsem1)
            wait_gathers(idxb0, rows0, gsem0)

            @pl.when(c0 + 2 < ncht)
            def _():
                issue_idx(c0 + 2, idxb0, isem0)
            compute(rows0, stage0, tbuf)
            issue_outs(c0, stage0, osem0)

            @pl.when(c1 >= 2)
            def _():
                wait_outs(c1 - 2, stage1, osem1)

            @pl.when(c1 + 1 < ncht)
            def _():
                wait_idx(idxb0, isem0)
                issue_gathers(idxb0, rows0, gsem0)
            wait_gathers(idxb1, rows1, gsem1)

            @pl.when(c1 + 2 < ncht)
            def _():
                issue_idx(c1 + 2, idxb1, isem1)
            compute(rows1, stage1, tbuf)
            issue_outs(c1, stage1, osem1)
            return carry

        lax.fori_loop(0, npair, pair_body, 0)
        wait_outs(ncht - 2, stage0, osem0)
        wait_outs(ncht - 1, stage1, osem1)

    return k, per_w, ncht, nreg


def kernel(z, pos_edge_index, n_nodes):
    n_edges = pos_edge_index.shape[1]
    neg_dst = jax.random.randint(
        jax.random.key(42), (n_edges * N_NEG,), 0, n_nodes, dtype=jnp.int32)
    src = pos_edge_index[0]
    dst = pos_edge_index[1]
    k, per_w, ncht, nreg = _make_sc_kernel(n_edges)
    idx_pack = _pack_indices(src, dst, neg_dst, per_w, ncht, nreg)
    pos_scores, neg_scores = k(z, idx_pack)
    return (pos_scores, neg_scores)


# speedup vs baseline: 8.2402x; 1.3891x over previous
import functools

import jax
import jax.numpy as jnp
from jax import lax
from jax.experimental import pallas as pl
from jax.experimental.pallas import tpu as pltpu
from jax.experimental.pallas import tpu_sc as plsc

N_NEG = 5
D = 128
LANES = 16
CHUNK = 64
NCHUNK = N_NEG * CHUNK
ROWS = 2 * CHUNK + NCHUNK
GROUPS = CHUNK // LANES
IB = 8
FUNROLL = 16
NC = 2
NS = 16
NW = NC * NS

_SEGS = []
_off = 0
while _off < ROWS:
    _len = min(128, ROWS - _off)
    _SEGS.append((_off, _len))
    _off += _len


def _pack_indices(src, dst, neg_dst, per_w, ncht, nreg):
    def chunkify(a, cw):
        a = a.reshape(NW, -1)
        reg = a[:, :nreg * cw].reshape(NW, nreg, cw)
        parts = [reg]
        if ncht > nreg:
            tail = a[:, a.shape[1] - cw:][:, None, :]
            parts.append(jnp.broadcast_to(tail, (NW, ncht - nreg, cw)))
        return jnp.concatenate(parts, axis=1) if len(parts) > 1 else reg

    return jnp.concatenate(
        [chunkify(src, CHUNK), chunkify(dst, CHUNK),
         chunkify(neg_dst, NCHUNK)], axis=2)


def _make_sc_kernel(n_edges):
    per_w = n_edges // NW
    assert per_w * NW == n_edges
    assert per_w >= CHUNK and (per_w - CHUNK) % 8 == 0
    nreg = per_w // CHUNK
    covered = nreg * CHUNK
    ncht = ((nreg + (1 if covered < per_w else 0) + IB - 1) // IB) * IB
    npair = ncht // 2
    tail_start = per_w - CHUNK

    mesh = plsc.VectorSubcoreMesh(
        core_axis_name="c", subcore_axis_name="s",
        num_cores=NC, num_subcores=NS)

    @functools.partial(
        pl.kernel,
        mesh=mesh,
        out_type=(
            jax.ShapeDtypeStruct((n_edges,), jnp.float32),
            jax.ShapeDtypeStruct((N_NEG * n_edges,), jnp.float32),
        ),
        scratch_types=(
            pltpu.VMEM((ROWS,), jnp.int32),
            pltpu.VMEM((ROWS,), jnp.int32),
            pltpu.VMEM((ROWS, D), jnp.float32),
            pltpu.VMEM((ROWS, D), jnp.float32),
            pltpu.VMEM((CHUNK + NCHUNK,), jnp.float32),
            pltpu.VMEM((CHUNK + NCHUNK,), jnp.float32),
            pltpu.VMEM(((1 + N_NEG) * LANES, LANES), jnp.float32),
            pltpu.SemaphoreType.DMA,
            pltpu.SemaphoreType.DMA,
            pltpu.SemaphoreType.DMA,
            pltpu.SemaphoreType.DMA,
            pltpu.SemaphoreType.DMA,
            pltpu.SemaphoreType.DMA,
        ),
        compiler_params=pltpu.CompilerParams(needs_layout_passes=False),
    )
    def k(z_hbm, idx_hbm, pos_out, neg_out,
          idxb0, idxb1, rows0, rows1, stage0, stage1, tbuf,
          gsem0, gsem1, osem0, osem1, isem0, isem1):
        wid = lax.axis_index("s") * NC + lax.axis_index("c")
        wbase = wid * per_w

        def issue_gathers(idxb_q, rows_q, gsem_q):
            for off, ln in _SEGS:
                pltpu.async_copy(
                    z_hbm.at[idxb_q.at[pl.ds(off, ln)]],
                    rows_q.at[pl.ds(off, ln)], gsem_q)

        def wait_gathers(idxb_q, rows_q, gsem_q):
            for off, ln in _SEGS:
                pltpu.make_async_copy(
                    z_hbm.at[idxb_q.at[pl.ds(off, ln)]],
                    rows_q.at[pl.ds(off, ln)], gsem_q).wait()

        def issue_idx(c, idxb_q, isem_q):
            pltpu.async_copy(idx_hbm.at[wid, c], idxb_q, isem_q)

        def wait_idx(idxb_q, isem_q):
            pltpu.make_async_copy(idx_hbm.at[wid, 0], idxb_q, isem_q).wait()

        def wait_outs(c, stage_q, osem_q):
            gb = pl.multiple_of(
                wbase + jnp.minimum(c * CHUNK, tail_start), 8)
            pltpu.make_async_copy(
                stage_q.at[pl.ds(0, CHUNK)],
                pos_out.at[pl.ds(gb, CHUNK)], osem_q).wait()
            pltpu.make_async_copy(
                stage_q.at[pl.ds(CHUNK, NCHUNK)],
                neg_out.at[pl.ds(gb * N_NEG, NCHUNK)], osem_q).wait()

        def compute(rows_q, stage_q, tb):
            iota = lax.iota(jnp.int32, LANES)

            def group_body(g, gc):
                ebase = g * LANES

                def edge_body(il, ec):
                    for u in range(4):
                        el = il * 4 + u
                        e = ebase + el
                        prows = [CHUNK + e] + [
                            2 * CHUNK + N_NEG * e + kk
                            for kk in range(N_NEG)]
                        partials = [None] * (1 + N_NEG)
                        for fb in range(4):
                            f0 = 32 * fb
                            s2 = [rows_q[e, pl.ds(f0 + 16 * j, 16)]
                                  for j in range(2)]
                            for pi, prow in enumerate(prows):
                                m0 = s2[0] * rows_q[prow, pl.ds(f0, 16)]
                                m1 = s2[1] * rows_q[prow, pl.ds(f0 + 16, 16)]
                                t = m0 + m1
                                partials[pi] = (
                                    t if fb == 0 else partials[pi] + t)
                        tb[el, :] = partials[0]
                        for kk in range(N_NEG):
                            tb[(1 + kk) * LANES + el, :] = partials[1 + kk]
                    return ec

                lax.fori_loop(0, LANES // 4, edge_body, 0)

                def colsum(b):
                    rvec = jnp.full((LANES,), b * LANES, jnp.int32) + iota
                    gs = [plsc.load_gather(
                        tb, [rvec, jnp.full((LANES,), j, jnp.int32)])
                        for j in range(LANES)]
                    for step in (8, 4, 2, 1):
                        gs = [gs[i] + gs[i + step] for i in range(step)]
                    return gs[0]

                stage_q[pl.ds(ebase, LANES)] = (
                    1.0 / (1.0 + jnp.exp(-colsum(0))))
                nidx0 = CHUNK + (ebase + iota) * N_NEG
                for kk in range(N_NEG):
                    plsc.store_scatter(
                        stage_q, [nidx0 + kk],
                        1.0 / (1.0 + jnp.exp(-colsum(1 + kk))))
                return gc

            lax.fori_loop(0, GROUPS, group_body, 0)

        def issue_outs(c, stage_q, osem_q):
            gb = pl.multiple_of(
                wbase + jnp.minimum(c * CHUNK, tail_start), 8)
            pltpu.async_copy(stage_q.at[pl.ds(0, CHUNK)],
                             pos_out.at[pl.ds(gb, CHUNK)], osem_q)
            pltpu.async_copy(stage_q.at[pl.ds(CHUNK, NCHUNK)],
                             neg_out.at[pl.ds(gb * N_NEG, NCHUNK)], osem_q)

        pltpu.sync_copy(idx_hbm.at[wid, 0], idxb0)
        issue_gathers(idxb0, rows0, gsem0)
        issue_idx(1, idxb1, isem1)

        def pair_body(i, carry):
            c0 = 2 * i
            c1 = c0 + 1

            @pl.when(c0 >= 2)
            def _():
                wait_outs(c0 - 2, stage0, osem0)
            wait_idx(idxb1, isem1)
            issue_gathers(idxb1, rows1, gsem1)
            wait_gathers(idxb0, rows0, gsem0)

            @pl.when(c0 + 2 < ncht)
            def _():
                issue_idx(c0 + 2, idxb0, isem0)
            compute(rows0, stage0, tbuf)
            issue_outs(c0, stage0, osem0)

            @pl.when(c1 >= 2)
            def _():
                wait_outs(c1 - 2, stage1, osem1)

            @pl.when(c1 + 1 < ncht)
            def _():
                wait_idx(idxb0, isem0)
                issue_gathers(idxb0, rows0, gsem0)
            wait_gathers(idxb1, rows1, gsem1)

            @pl.when(c1 + 2 < ncht)
            def _():
                issue_idx(c1 + 2, idxb1, isem1)
            compute(rows1, stage1, tbuf)
            issue_outs(c1, stage1, osem1)
            return carry

        lax.fori_loop(0, npair, pair_body, 0)
        wait_outs(ncht - 2, stage0, osem0)
        wait_outs(ncht - 1, stage1, osem1)

    return k, per_w, ncht, nreg


def kernel(z, pos_edge_index, n_nodes):
    n_edges = pos_edge_index.shape[1]
    neg_dst = jax.random.randint(
        jax.random.key(42), (n_edges * N_NEG,), 0, n_nodes, dtype=jnp.int32)
    src = pos_edge_index[0]
    dst = pos_edge_index[1]
    k, per_w, ncht, nreg = _make_sc_kernel(n_edges)
    idx_pack = _pack_indices(src, dst, neg_dst, per_w, ncht, nreg)
    pos_scores, neg_scores = k(z, idx_pack)
    return (pos_scores, neg_scores)
